# Initial kernel scaffold; baseline (speedup 1.0000x reference)
import jax
import jax.numpy as jnp
from jax.experimental import pallas as pl


def kernel(user_feat, item_feat, edge_ui, edge_iu, edge_uu, params):
    raise NotImplementedError("write your pallas kernel here")


# speedup vs baseline: 2.7544x; 2.7544x over previous
import functools

import jax
import jax.numpy as jnp
from jax.experimental import pallas as pl
from jax.experimental.pallas import tpu as pltpu

NU = 10000
NI = 10000
NE = 320000
D = 128

ROWS_BLK = 1000


def _proj_body(u_ref, i_ref, w_ref, b_ref, *out_refs):
    u = u_ref[...]
    it = i_ref[...]
    srcs = (u, it, it, u, u, u)
    for k, (x, o) in enumerate(zip(srcs, out_refs)):
        o[...] = jnp.dot(x, w_ref[k], preferred_element_type=jnp.float32) + b_ref[k]


def _projections(user_feat, item_feat, params):
    Ws = jnp.stack([
        params['gat_ui']['W_src'], params['gat_ui']['W_dst'],
        params['gat_iu']['W_src'], params['gat_iu']['W_dst'],
        params['gat_uu']['W_src'], params['gat_uu']['W_dst'],
    ])
    bs = jnp.stack([
        params['gat_ui']['b_src'], params['gat_ui']['b_dst'],
        params['gat_iu']['b_src'], params['gat_iu']['b_dst'],
        params['gat_uu']['b_src'], params['gat_uu']['b_dst'],
    ])
    grid = NU // ROWS_BLK
    blk = pl.BlockSpec((ROWS_BLK, D), lambda i: (i, 0))
    full_w = pl.BlockSpec((6, D, D), lambda i: (0, 0, 0))
    full_b = pl.BlockSpec((6, D), lambda i: (0, 0))
    outs = pl.pallas_call(
        _proj_body,
        grid=(grid,),
        in_specs=[blk, blk, full_w, full_b],
        out_specs=[blk] * 6,
        out_shape=[jax.ShapeDtypeStruct((NU, D), jnp.float32)] * 6,
    )(user_feat, item_feat, Ws, bs)
    return outs


def _edge_stage(h_src, h_dst, attn, edge, n_dst):
    u, v = edge[0], edge[1]
    e = jax.nn.leaky_relu(h_src[u] + h_dst[v], 0.2)
    score = e @ attn
    p = jnp.exp(score)
    denom = jax.ops.segment_sum(p, v, num_segments=n_dst)
    accum = jax.ops.segment_sum(p[:, None] * h_src[u], v, num_segments=n_dst)
    return accum / (denom[:, None] + 1e-9)


def _combine_body(u_ref, p_ref, q_ref, a_ref, c_ref, g_ref, out_ref):
    u = u_ref[...]
    ph = p_ref[...]
    qh = q_ref[...]
    raw_u = jnp.dot(u, a_ref[...], preferred_element_type=jnp.float32)
    rp = jnp.sum(ph * c_ref[0, 0:D], axis=1, keepdims=True)
    rq = jnp.sum(qh * c_ref[0, D:2 * D], axis=1, keepdims=True)
    raw = raw_u + jnp.concatenate([rp, rq], axis=1) + c_ref[0, 2 * D:2 * D + 2]
    mean = jnp.mean(raw, axis=0, keepdims=True)
    var = jnp.mean((raw - mean) ** 2, axis=0, keepdims=True)
    g = jnp.concatenate([g_ref[0, 0:1], g_ref[0, 2:3]])
    b = jnp.concatenate([g_ref[0, 1:2], g_ref[0, 3:4]])
    h = (raw - mean) / jnp.sqrt(var + 1e-5) * g + b
    h = jnp.where(h >= 0, h, 0.01 * h)
    m = jnp.max(h, axis=1, keepdims=True)
    eh = jnp.exp(h - m)
    gam = eh / jnp.sum(eh, axis=1, keepdims=True)
    out_ref[...] = gam[:, 0:1] * ph + gam[:, 1:2] * qh + u


def _combine(user_feat, p_hair, q_hair, params):
    pi, pt = params['att_influence'], params['att_interest']
    we_i = pi['W1'] @ pi['W2']
    we_t = pt['W1'] @ pt['W2']
    bias_i = pi['b1'] @ pi['W2'] + pi['b2']
    bias_t = pt['b1'] @ pt['W2'] + pt['b2']
    A = jnp.concatenate([we_i[:D], we_t[:D]], axis=1)
    c = jnp.concatenate([we_i[D:, 0], we_t[D:, 0], bias_i, bias_t,
                         jnp.zeros((2 * D + 2,), jnp.float32)])[None, :2 * D + 8]
    c = c[:, :2 * D + 2]
    c = jnp.pad(c, ((0, 0), (0, 8 * ((2 * D + 2 + 7) // 8) - (2 * D + 2))))
    g = jnp.stack([pi['bn_gamma'][0], pi['bn_beta'][0],
                   pt['bn_gamma'][0], pt['bn_beta'][0],
                   0.0, 0.0, 0.0, 0.0])[None, :]
    full = lambda shape: pl.BlockSpec(shape, lambda: tuple(0 for _ in shape))
    return pl.pallas_call(
        _combine_body,
        in_specs=[full((NU, D)), full((NU, D)), full((NU, D)),
                  full((D, 2)), full(c.shape), full((1, 8))],
        out_specs=full((NU, D)),
        out_shape=jax.ShapeDtypeStruct((NU, D), jnp.float32),
    )(user_feat, p_hair, q_hair, A, c, g)


def kernel(user_feat, item_feat, edge_ui, edge_iu, edge_uu, params):
    h_ui_s, h_ui_d, h_iu_s, h_iu_d, h_uu_s, h_uu_d = _projections(
        user_feat, item_feat, params)
    item_agg = _edge_stage(h_ui_s, h_ui_d, params['gat_ui']['attn'], edge_ui, NI)
    q_hair = _edge_stage(h_iu_s, h_iu_d, params['gat_iu']['attn'], edge_iu, NU)
    p_hair = _edge_stage(h_uu_s, h_uu_d, params['gat_uu']['attn'], edge_uu, NU)
    item_emb = item_agg + item_feat
    user_emb = _combine(user_feat, p_hair, q_hair, params)
    return (user_emb, item_emb)


# speedup vs baseline: 3.3272x; 1.2079x over previous
import functools

import jax
import jax.numpy as jnp
from jax import lax
from jax.experimental import pallas as pl
from jax.experimental.pallas import tpu as pltpu
from jax.experimental.pallas import tpu_sc as plsc

NU = 10000
NE = 320000
D = 128

NP = 10112
RPT = NP // 16
DUMMY = 10008
CHUNK = 64
NW = 32
EPT = 10112
NEP = NW * EPT
NCHUNK = EPT // CHUNK

ROWS_BLK = 1264


def _proj_body(u_ref, i_ref, w_ref, b_ref, *out_refs):
    u = u_ref[...]
    it = i_ref[...]
    srcs = (u, it, it, u, u, u)
    for k, (x, o) in enumerate(zip(srcs, out_refs)):
        o[...] = jnp.dot(x, w_ref[k], preferred_element_type=jnp.float32) + b_ref[k]


def _projections(user_pad, item_pad, params):
    Ws = jnp.stack([
        params['gat_ui']['W_src'], params['gat_ui']['W_dst'],
        params['gat_iu']['W_src'], params['gat_iu']['W_dst'],
        params['gat_uu']['W_src'], params['gat_uu']['W_dst'],
    ])
    bs = jnp.stack([
        params['gat_ui']['b_src'], params['gat_ui']['b_dst'],
        params['gat_iu']['b_src'], params['gat_iu']['b_dst'],
        params['gat_uu']['b_src'], params['gat_uu']['b_dst'],
    ])
    grid = NP // ROWS_BLK
    blk = pl.BlockSpec((ROWS_BLK, D), lambda i: (i, 0))
    full_w = pl.BlockSpec((6, D, D), lambda i: (0, 0, 0))
    full_b = pl.BlockSpec((6, D), lambda i: (0, 0))
    return pl.pallas_call(
        _proj_body,
        grid=(grid,),
        in_specs=[blk, blk, full_w, full_b],
        out_specs=[blk] * 6,
        out_shape=[jax.ShapeDtypeStruct((NP, D), jnp.float32)] * 6,
    )(user_pad, item_pad, Ws, bs)


def _lane_shuffle(x, perm):
    dn = lax.GatherDimensionNumbers(
        offset_dims=(), collapsed_slice_dims=(0,), start_index_map=(0,))
    return lax.gather(x, perm[:, None], dimension_numbers=dn,
                      slice_sizes=(1,),
                      mode=lax.GatherScatterMode.PROMISE_IN_BOUNDS)


def _lane_allsum(x):
    idx = lax.iota(jnp.int32, 16)
    for k in (1, 2, 4, 8):
        x = x + _lane_shuffle(x, jnp.bitwise_xor(idx, k))
    return x


def _edge_sc_body(hs_hbm, hd_hbm, attn_hbm, u_hbm, v_hbm, rowids_hbm,
                  outw_hbm, outd_hbm,
                  u_idx, v_idx, hs, hd, pvb, pvb128, attn_v, accw,
                  sem1, sem2):
    c = lax.axis_index("c")
    s = lax.axis_index("s")
    wid = s * 2 + c
    r0 = pl.multiple_of(s * RPT, 8)
    NBLK = (RPT + CHUNK - 1) // CHUNK
    zv = jnp.zeros((16,), jnp.float32)

    def zrow(i, carry):
        for j in range(8):
            hs[i, pl.ds(16 * j, 16)] = zv
        return carry

    def set_rowidx(ro):
        pltpu.sync_copy(rowids_hbm.at[pl.ds(ro, CHUNK)], u_idx)

    def zinit(k, carry):
        off = pl.multiple_of(jnp.minimum(k * CHUNK, RPT - CHUNK), 8)
        set_rowidx(r0 + off)
        pltpu.sync_copy(hs, accw.at[u_idx])
        return carry

    def dmp(out_hbm):
        o0 = pl.multiple_of(c * NP + r0, 8)

        def dmp_k(k, carry):
            off = pl.multiple_of(jnp.minimum(k * CHUNK, RPT - CHUNK), 8)
            set_rowidx(r0 + off)
            pltpu.async_copy(accw.at[u_idx], hs, sem1).wait()
            oo = pl.multiple_of(o0 + off, 8)
            pltpu.sync_copy(hs, out_hbm.at[pl.ds(oo, CHUNK)])
            return carry

        lax.fori_loop(0, NBLK, dmp_k, 0)

    lax.fori_loop(0, CHUNK, zrow, 0)
    lax.fori_loop(0, NBLK, zinit, 0)
    pltpu.sync_copy(attn_hbm, attn_v)
    a_regs = [attn_v[pl.ds(16 * j, 16)] for j in range(8)]
    plsc.subcore_barrier()

    base0 = wid * EPT

    def make_chunk_body(weighted):
        def chunk_body(ci, carry):
            base = pl.multiple_of(base0 + ci * CHUNK, CHUNK)
            pltpu.sync_copy(u_hbm.at[pl.ds(base, CHUNK)], u_idx)
            pltpu.sync_copy(v_hbm.at[pl.ds(base, CHUNK)], v_idx)
            cp1 = pltpu.async_copy(hs_hbm.at[u_idx], hs, sem1)
            cp2 = pltpu.async_copy(hd_hbm.at[v_idx], hd, sem2)
            cp1.wait()
            cp2.wait()

            def edge_body(e, carry2):
                hsv = [hs[e, pl.ds(16 * j, 16)] for j in range(8)]
                acc = jnp.zeros((16,), jnp.float32)
                for j in range(8):
                    t = hsv[j] + hd[e, pl.ds(16 * j, 16)]
                    lr = t * 0.6 + jnp.abs(t) * 0.4
                    acc = acc + lr * a_regs[j]
                pv = jnp.exp(_lane_allsum(acc))
                if weighted:
                    for j in range(8):
                        hs[e, pl.ds(16 * j, 16)] = pv * hsv[j]
                else:
                    for j in range(8):
                        pvb128[e, pl.ds(16 * j, 16)] = pv
                return carry2

            lax.fori_loop(0, CHUNK, edge_body, 0, unroll=2)
            src = hs if weighted else pvb128
            pltpu.sync_copy(src, accw.at[v_idx], add=True)
            return carry
        return chunk_body

    lax.fori_loop(0, NCHUNK, make_chunk_body(True), 0)
    plsc.subcore_barrier()
    dmp(outw_hbm)
    lax.fori_loop(0, CHUNK, zrow, 0)
    lax.fori_loop(0, NBLK, zinit, 0)
    plsc.subcore_barrier()

    lax.fori_loop(0, NCHUNK, make_chunk_body(False), 0)
    plsc.subcore_barrier()
    dmp(outd_hbm)


def _edge_sc(h_src, h_dst, attn, u, v):
    mesh = plsc.VectorSubcoreMesh(core_axis_name="c", subcore_axis_name="s")
    f = functools.partial(
        pl.kernel,
        out_type=[jax.ShapeDtypeStruct((2 * NP, D), jnp.float32),
                  jax.ShapeDtypeStruct((2 * NP, D), jnp.float32)],
        mesh=mesh,
        scratch_types=[
            pltpu.VMEM((CHUNK,), jnp.int32),
            pltpu.VMEM((CHUNK,), jnp.int32),
            pltpu.VMEM((CHUNK, D), jnp.float32),
            pltpu.VMEM((CHUNK, D), jnp.float32),
            pltpu.VMEM((CHUNK, 16), jnp.float32),
            pltpu.VMEM((CHUNK, D), jnp.float32),
            pltpu.VMEM((D,), jnp.float32),
            pltpu.VMEM_SHARED((NP, D), jnp.float32),
            pltpu.SemaphoreType.DMA,
            pltpu.SemaphoreType.DMA,
        ],
    )(_edge_sc_body)
    rowids = jnp.arange(NP, dtype=jnp.int32)
    aw, ad = f(h_src, h_dst, attn, u, v, rowids)
    return aw.reshape(2, NP, D), ad.reshape(2, NP, D)


def _norm_body(aw_ref, ad_ref, res_ref, out_ref):
    aw = aw_ref[0] + aw_ref[1]
    den = ad_ref[0, :, 0:1] + ad_ref[1, :, 0:1]
    out_ref[...] = aw[:NU] / (den[:NU] + 1e-9) + res_ref[...]


def _normalize(accw, accd, residual):
    full = lambda shape: pl.BlockSpec(shape, lambda: tuple(0 for _ in shape))
    return pl.pallas_call(
        _norm_body,
        in_specs=[full((2, NP, D)), full((2, NP, D)), full((NU, D))],
        out_specs=full((NU, D)),
        out_shape=jax.ShapeDtypeStruct((NU, D), jnp.float32),
    )(accw, accd, residual)


def _gathernorm_body(u_ref, awp_ref, adp_ref, awq_ref, adq_ref,
                     a_ref, c_ref, ph_ref, qh_ref, raw_ref):
    ph = ((awp_ref[0] + awp_ref[1])
          / (adp_ref[0, :, 0:1] + adp_ref[1, :, 0:1] + 1e-9))
    qh = ((awq_ref[0] + awq_ref[1])
          / (adq_ref[0, :, 0:1] + adq_ref[1, :, 0:1] + 1e-9))
    ph_ref[...] = ph
    qh_ref[...] = qh
    raw_u = jnp.dot(u_ref[...], a_ref[...], preferred_element_type=jnp.float32)
    rp = jnp.sum(ph * c_ref[0, 0:D], axis=1, keepdims=True)
    rq = jnp.sum(qh * c_ref[0, D:2 * D], axis=1, keepdims=True)
    raw_ref[...] = (raw_u + jnp.concatenate([rp, rq], axis=1)
                    + c_ref[0, 2 * D:2 * D + 2])


def _final_body(u_ref, ph_ref, qh_ref, raw_ref, g_ref, out_ref):
    u = u_ref[...]
    ph = ph_ref[:NU]
    qh = qh_ref[:NU]
    raw = raw_ref[:NU]
    mean = jnp.mean(raw, axis=0, keepdims=True)
    var = jnp.mean((raw - mean) ** 2, axis=0, keepdims=True)
    g = jnp.concatenate([g_ref[0, 0:1], g_ref[0, 2:3]])
    b = jnp.concatenate([g_ref[0, 1:2], g_ref[0, 3:4]])
    h = (raw - mean) / jnp.sqrt(var + 1e-5) * g + b
    h = jnp.where(h >= 0, h, 0.01 * h)
    m = jnp.max(h, axis=1, keepdims=True)
    eh = jnp.exp(h - m)
    gam = eh / jnp.sum(eh, axis=1, keepdims=True)
    out_ref[...] = gam[:, 0:1] * ph + gam[:, 1:2] * qh + u


def _combine(user_feat, user_pad, accw_uu, accd_uu, accw_iu, accd_iu, params):
    pi, pt = params['att_influence'], params['att_interest']
    we_i = pi['W1'] @ pi['W2']
    we_t = pt['W1'] @ pt['W2']
    bias_i = pi['b1'] @ pi['W2'] + pi['b2']
    bias_t = pt['b1'] @ pt['W2'] + pt['b2']
    A = jnp.concatenate([we_i[:D], we_t[:D]], axis=1)
    c = jnp.concatenate([we_i[D:, 0], we_t[D:, 0], bias_i, bias_t,
                         jnp.zeros((6,), jnp.float32)])[None, :]
    g = jnp.stack([pi['bn_gamma'][0], pi['bn_beta'][0],
                   pt['bn_gamma'][0], pt['bn_beta'][0],
                   0.0, 0.0, 0.0, 0.0])[None, :]
    RB = 1264
    grid = NP // RB
    rows = pl.BlockSpec((RB, D), lambda i: (i, 0))
    acc_w = pl.BlockSpec((2, RB, D), lambda i: (0, i, 0))
    acc_d = pl.BlockSpec((2, RB, D), lambda i: (0, i, 0))
    small = lambda shape: pl.BlockSpec(shape, lambda i: tuple(0 for _ in shape))
    ph, qh, raw = pl.pallas_call(
        _gathernorm_body,
        grid=(grid,),
        in_specs=[rows, acc_w, acc_d, acc_w, acc_d,
                  small((D, 2)), small(c.shape)],
        out_specs=[rows, rows, pl.BlockSpec((RB, 2), lambda i: (i, 0))],
        out_shape=[jax.ShapeDtypeStruct((NP, D), jnp.float32),
                   jax.ShapeDtypeStruct((NP, D), jnp.float32),
                   jax.ShapeDtypeStruct((NP, 2), jnp.float32)],
    )(user_pad, accw_uu, accd_uu, accw_iu, accd_iu, A, c)
    full = lambda shape: pl.BlockSpec(shape, lambda: tuple(0 for _ in shape))
    return pl.pallas_call(
        _final_body,
        in_specs=[full((NU, D)), full((NP, D)), full((NP, D)),
                  full((NP, 2)), full((1, 8))],
        out_specs=full((NU, D)),
        out_shape=jax.ShapeDtypeStruct((NU, D), jnp.float32),
    )(user_feat, ph, qh, raw, g)


def kernel(user_feat, item_feat, edge_ui, edge_iu, edge_uu, params):
    user_pad = jnp.pad(user_feat, ((0, NP - NU), (0, 0)))
    item_pad = jnp.pad(item_feat, ((0, NP - NU), (0, 0)))
    h_ui_s, h_ui_d, h_iu_s, h_iu_d, h_uu_s, h_uu_d = _projections(
        user_pad, item_pad, params)

    pad = jnp.full((NEP - NE,), DUMMY, jnp.int32)
    edges = {}
    for name, e in (('ui', edge_ui), ('iu', edge_iu), ('uu', edge_uu)):
        edges[name] = (jnp.concatenate([e[0], pad]),
                       jnp.concatenate([e[1], pad]))

    aw_ui, ad_ui = _edge_sc(h_ui_s, h_ui_d, params['gat_ui']['attn'],
                            edges['ui'][0], edges['ui'][1])
    aw_iu, ad_iu = _edge_sc(h_iu_s, h_iu_d, params['gat_iu']['attn'],
                            edges['iu'][0], edges['iu'][1])
    aw_uu, ad_uu = _edge_sc(h_uu_s, h_uu_d, params['gat_uu']['attn'],
                            edges['uu'][0], edges['uu'][1])

    item_emb = _normalize(aw_ui, ad_ui, item_feat)
    user_emb = _combine(user_feat, user_pad, aw_uu, ad_uu, aw_iu, ad_iu, params)
    return (user_emb, item_emb)
